# baseline (speedup 1.0000x reference)
import functools

import jax
import jax.numpy as jnp
from jax.experimental import pallas as pl
from jax.experimental.pallas import tpu as pltpu

_K = 16
_N = 4096
_R = 256
_NBLK = _N // _R


def _topk_mask(D):
    big = jnp.float32(1e30)

    def it(t, carry):
        D, M = carry
        m = jnp.min(D, axis=1, keepdims=True)
        hb = D == m
        return jnp.where(hb, big, D), M + hb.astype(jnp.float32)

    D, M = jax.lax.fori_loop(0, _K - 1, it, (D, jnp.zeros(D.shape, jnp.float32)))
    m = jnp.min(D, axis=1, keepdims=True)
    return M + (D == m).astype(jnp.float32)


def _acos(x):
    ax = jnp.abs(x)
    p = jnp.float32(-0.0012624911)
    p = p * ax + jnp.float32(0.0066700901)
    p = p * ax + jnp.float32(-0.0170881256)
    p = p * ax + jnp.float32(0.0308918810)
    p = p * ax + jnp.float32(-0.0501743046)
    p = p * ax + jnp.float32(0.0889789874)
    p = p * ax + jnp.float32(-0.2145988016)
    p = p * ax + jnp.float32(1.5707963050)
    a_pos = jnp.sqrt(jnp.maximum(1.0 - ax, 0.0)) * p
    return jnp.where(x >= 0, a_pos, jnp.float32(3.14159265358979) - a_pos)


def _eigen_ratio_block(S):
    k = jnp.float32(_K)
    mx = S[:, 0:1] / k
    my = S[:, 1:2] / k
    mz = S[:, 2:3] / k
    cxx = S[:, 3:4] / k - mx * mx
    cyy = S[:, 4:5] / k - my * my
    czz = S[:, 5:6] / k - mz * mz
    cxy = S[:, 6:7] / k - mx * my
    cxz = S[:, 7:8] / k - mx * mz
    cyz = S[:, 8:9] / k - my * mz

    q = (cxx + cyy + czz) * jnp.float32(1.0 / 3.0)
    p1 = cxy * cxy + cxz * cxz + cyz * cyz
    dxx = cxx - q
    dyy = cyy - q
    dzz = czz - q
    p2 = dxx * dxx + dyy * dyy + dzz * dzz + 2.0 * p1
    eps = jnp.float32(1e-30)
    safe = p2 > eps
    p = jnp.sqrt(jnp.maximum(p2, eps) * jnp.float32(1.0 / 6.0))
    inv_p = 1.0 / p
    b00 = dxx * inv_p
    b11 = dyy * inv_p
    b22 = dzz * inv_p
    b01 = cxy * inv_p
    b02 = cxz * inv_p
    b12 = cyz * inv_p
    detb = (b00 * (b11 * b22 - b12 * b12)
            - b01 * (b01 * b22 - b12 * b02)
            + b02 * (b01 * b12 - b11 * b02))
    r = jnp.clip(detb * 0.5, -1.0, 1.0)
    phi = _acos(r) * jnp.float32(1.0 / 3.0)
    e0 = q + 2.0 * p * jnp.cos(phi)
    e2 = q + 2.0 * p * jnp.cos(phi + jnp.float32(2.0943951023931953))
    e1 = 3.0 * q - e0 - e2
    return jnp.where(safe, e0 / e1, jnp.float32(1.0))


def _body(px_ref, pxT_ref, py_ref, pyT_ref, out_ref, er1_ref, er2_ref):
    b = pl.program_id(0)

    @pl.when(b == 0)
    def _():
        out_ref[:, :] = jnp.zeros((1, 1), jnp.float32)

    iota_i32 = jax.lax.broadcasted_iota(jnp.int32, (_R, _N), 1)

    def er_phase(p_ref, pT_ref, er_ref):
        pT = pT_ref[0]
        pts2 = jnp.sum(pT * pT, axis=0, keepdims=True)
        pT16 = pT.astype(jnp.bfloat16)

        def blk(i, _):
            r0 = i * _R
            a = p_ref[0, pl.ds(r0, _R), 0:3].astype(jnp.bfloat16)
            D = pts2 - 2.0 * jnp.dot(a, pT16, preferred_element_type=jnp.float32)
            M = _topk_mask(D)
            S = jnp.dot(M, p_ref[0], preferred_element_type=jnp.float32, precision=jax.lax.Precision.HIGHEST)
            er_ref[pl.ds(r0, _R), :] = _eigen_ratio_block(S)
            return 0

        jax.lax.fori_loop(0, _NBLK, blk, 0)

    er_phase(px_ref, pxT_ref, er1_ref)
    er_phase(py_ref, pyT_ref, er2_ref)

    def cross_phase(pa_ref, pbT_ref, era_ref, erb_ref):
        pbT = pbT_ref[0]
        pts2 = jnp.sum(pbT * pbT, axis=0, keepdims=True)
        pbT16 = pbT.astype(jnp.bfloat16)
        erb = erb_ref[:, :]

        def blk(i, sse):
            r0 = i * _R
            a = pa_ref[0, pl.ds(r0, _R), 0:3].astype(jnp.bfloat16)
            D = pts2 - 2.0 * jnp.dot(a, pbT16, preferred_element_type=jnp.float32)
            m = jnp.min(D, axis=1, keepdims=True)
            idxv = jnp.where(D == m, iota_i32, jnp.int32(2 * _N))
            amin = jnp.min(idxv, axis=1, keepdims=True)
            h = (iota_i32 == amin).astype(jnp.float32)
            corr = jnp.dot(h, erb, preferred_element_type=jnp.float32, precision=jax.lax.Precision.HIGHEST)
            d = era_ref[pl.ds(r0, _R), :] - corr
            return sse + jnp.sum(d * d, axis=(0, 1), keepdims=True)

        return jax.lax.fori_loop(0, _NBLK, blk, jnp.zeros((1, 1), jnp.float32))

    sse_x = cross_phase(px_ref, pyT_ref, er1_ref, er2_ref)
    sse_y = cross_phase(py_ref, pxT_ref, er2_ref, er1_ref)
    out_ref[:, :] += sse_x + sse_y


def _moments(pts):
    x = pts[..., 0:1]
    y = pts[..., 1:2]
    z = pts[..., 2:3]
    zeros = jnp.zeros(pts.shape[:-1] + (7,), pts.dtype)
    return jnp.concatenate(
        [x, y, z, x * x, y * y, z * z, x * y, x * z, y * z, zeros], axis=-1)


@jax.jit
def kernel(x, y):
    x3 = x[..., :3].astype(jnp.float32)
    y3 = y[..., :3].astype(jnp.float32)
    px = _moments(x3)
    py = _moments(y3)
    pxT = jnp.swapaxes(x3, 1, 2)
    pyT = jnp.swapaxes(y3, 1, 2)

    bspec_p = pl.BlockSpec((1, _N, 16), lambda b: (b, 0, 0))
    bspec_t = pl.BlockSpec((1, 3, _N), lambda b: (b, 0, 0))
    acc = pl.pallas_call(
        _body,
        grid=(x.shape[0],),
        in_specs=[bspec_p, bspec_t, bspec_p, bspec_t],
        out_specs=pl.BlockSpec((1, 1), lambda b: (0, 0)),
        out_shape=jax.ShapeDtypeStruct((1, 1), jnp.float32),
        scratch_shapes=[pltpu.VMEM((_N, 1), jnp.float32),
                        pltpu.VMEM((_N, 1), jnp.float32)],
    )(px, pxT, py, pyT)
    return acc[0, 0] * jnp.float32(0.5 / (_N * x.shape[0]))


# speedup vs baseline: 1.9843x; 1.9843x over previous
import functools

import jax
import jax.numpy as jnp
from jax.experimental import pallas as pl
from jax.experimental.pallas import tpu as pltpu

_K = 16
_N = 4096
_R = 128
_NBLK = _N // _R


def _topk_mask(D):
    M = jnp.zeros(D.shape, jnp.float32)
    big = jnp.float32(1e30)
    for t in range(_K):
        m = jnp.min(D, axis=1, keepdims=True)
        hb = D == m
        M = M + hb.astype(jnp.float32)
        if t + 1 < _K:
            D = jnp.where(hb, big, D)
    return M


def _acos(x):
    ax = jnp.abs(x)
    p = jnp.float32(-0.0012624911)
    p = p * ax + jnp.float32(0.0066700901)
    p = p * ax + jnp.float32(-0.0170881256)
    p = p * ax + jnp.float32(0.0308918810)
    p = p * ax + jnp.float32(-0.0501743046)
    p = p * ax + jnp.float32(0.0889789874)
    p = p * ax + jnp.float32(-0.2145988016)
    p = p * ax + jnp.float32(1.5707963050)
    a_pos = jnp.sqrt(jnp.maximum(1.0 - ax, 0.0)) * p
    return jnp.where(x >= 0, a_pos, jnp.float32(3.14159265358979) - a_pos)


def _eigen_ratio_block(S):
    k = jnp.float32(_K)
    mx = S[:, 0:1] / k
    my = S[:, 1:2] / k
    mz = S[:, 2:3] / k
    cxx = S[:, 3:4] / k - mx * mx
    cyy = S[:, 4:5] / k - my * my
    czz = S[:, 5:6] / k - mz * mz
    cxy = S[:, 6:7] / k - mx * my
    cxz = S[:, 7:8] / k - mx * mz
    cyz = S[:, 8:9] / k - my * mz

    q = (cxx + cyy + czz) * jnp.float32(1.0 / 3.0)
    p1 = cxy * cxy + cxz * cxz + cyz * cyz
    dxx = cxx - q
    dyy = cyy - q
    dzz = czz - q
    p2 = dxx * dxx + dyy * dyy + dzz * dzz + 2.0 * p1
    eps = jnp.float32(1e-30)
    safe = p2 > eps
    p = jnp.sqrt(jnp.maximum(p2, eps) * jnp.float32(1.0 / 6.0))
    inv_p = 1.0 / p
    b00 = dxx * inv_p
    b11 = dyy * inv_p
    b22 = dzz * inv_p
    b01 = cxy * inv_p
    b02 = cxz * inv_p
    b12 = cyz * inv_p
    detb = (b00 * (b11 * b22 - b12 * b12)
            - b01 * (b01 * b22 - b12 * b02)
            + b02 * (b01 * b12 - b11 * b02))
    r = jnp.clip(detb * 0.5, -1.0, 1.0)
    phi = _acos(r) * jnp.float32(1.0 / 3.0)
    e0 = q + 2.0 * p * jnp.cos(phi)
    e2 = q + 2.0 * p * jnp.cos(phi + jnp.float32(2.0943951023931953))
    e1 = 3.0 * q - e0 - e2
    return jnp.where(safe, e0 / e1, jnp.float32(1.0))


def _body(px_ref, pxT_ref, py_ref, pyT_ref, out_ref, er1_ref, er2_ref):
    b = pl.program_id(0)

    @pl.when(b == 0)
    def _():
        out_ref[:, :] = jnp.zeros((1, 1), jnp.float32)

    iota_i32 = jax.lax.broadcasted_iota(jnp.int32, (_R, _N), 1)

    def er_phase(p_ref, pT_ref, er_ref):
        pT = pT_ref[0]
        pts2 = jnp.sum(pT * pT, axis=0, keepdims=True)
        pT16 = pT.astype(jnp.bfloat16)

        def blk(i, _):
            r0 = i * _R
            a = p_ref[0, pl.ds(r0, _R), 0:3].astype(jnp.bfloat16)
            D = pts2 - 2.0 * jnp.dot(a, pT16, preferred_element_type=jnp.float32)
            M = _topk_mask(D)
            S = jnp.dot(M, p_ref[0], preferred_element_type=jnp.float32, precision=jax.lax.Precision.HIGHEST)
            er_ref[pl.ds(r0, _R), :] = _eigen_ratio_block(S)
            return 0

        jax.lax.fori_loop(0, _NBLK, blk, 0)

    er_phase(px_ref, pxT_ref, er1_ref)
    er_phase(py_ref, pyT_ref, er2_ref)

    def cross_phase(pa_ref, pbT_ref, era_ref, erb_ref):
        pbT = pbT_ref[0]
        pts2 = jnp.sum(pbT * pbT, axis=0, keepdims=True)
        pbT16 = pbT.astype(jnp.bfloat16)
        erb = erb_ref[:, :]

        def blk(i, sse):
            r0 = i * _R
            a = pa_ref[0, pl.ds(r0, _R), 0:3].astype(jnp.bfloat16)
            D = pts2 - 2.0 * jnp.dot(a, pbT16, preferred_element_type=jnp.float32)
            m = jnp.min(D, axis=1, keepdims=True)
            idxv = jnp.where(D == m, iota_i32, jnp.int32(2 * _N))
            amin = jnp.min(idxv, axis=1, keepdims=True)
            h = (iota_i32 == amin).astype(jnp.float32)
            corr = jnp.dot(h, erb, preferred_element_type=jnp.float32, precision=jax.lax.Precision.HIGHEST)
            d = era_ref[pl.ds(r0, _R), :] - corr
            return sse + jnp.sum(d * d, axis=(0, 1), keepdims=True)

        return jax.lax.fori_loop(0, _NBLK, blk, jnp.zeros((1, 1), jnp.float32))

    sse_x = cross_phase(px_ref, pyT_ref, er1_ref, er2_ref)
    sse_y = cross_phase(py_ref, pxT_ref, er2_ref, er1_ref)
    out_ref[:, :] += sse_x + sse_y


def _moments(pts):
    x = pts[..., 0:1]
    y = pts[..., 1:2]
    z = pts[..., 2:3]
    zeros = jnp.zeros(pts.shape[:-1] + (7,), pts.dtype)
    return jnp.concatenate(
        [x, y, z, x * x, y * y, z * z, x * y, x * z, y * z, zeros], axis=-1)


@jax.jit
def kernel(x, y):
    x3 = x[..., :3].astype(jnp.float32)
    y3 = y[..., :3].astype(jnp.float32)
    px = _moments(x3)
    py = _moments(y3)
    pxT = jnp.swapaxes(x3, 1, 2)
    pyT = jnp.swapaxes(y3, 1, 2)

    bspec_p = pl.BlockSpec((1, _N, 16), lambda b: (b, 0, 0))
    bspec_t = pl.BlockSpec((1, 3, _N), lambda b: (b, 0, 0))
    acc = pl.pallas_call(
        _body,
        grid=(x.shape[0],),
        in_specs=[bspec_p, bspec_t, bspec_p, bspec_t],
        out_specs=pl.BlockSpec((1, 1), lambda b: (0, 0)),
        out_shape=jax.ShapeDtypeStruct((1, 1), jnp.float32),
        scratch_shapes=[pltpu.VMEM((_N, 1), jnp.float32),
                        pltpu.VMEM((_N, 1), jnp.float32)],
    )(px, pxT, py, pyT)
    return acc[0, 0] * jnp.float32(0.5 / (_N * x.shape[0]))


# speedup vs baseline: 2.5200x; 1.2700x over previous
import functools

import jax
import jax.numpy as jnp
from jax.experimental import pallas as pl
from jax.experimental.pallas import tpu as pltpu

_K = 16
_N = 4096
_R = 128
_NBLK = _N // _R


def _topk_mask(D):
    big = jnp.float32(1e30)
    for _ in range(_K):
        m = jnp.min(D, axis=1, keepdims=True)
        D = jnp.where(D == m, big, D)
    return (D >= jnp.float32(1e29)).astype(jnp.float32)


def _acos(x):
    ax = jnp.abs(x)
    p = jnp.float32(-0.0012624911)
    p = p * ax + jnp.float32(0.0066700901)
    p = p * ax + jnp.float32(-0.0170881256)
    p = p * ax + jnp.float32(0.0308918810)
    p = p * ax + jnp.float32(-0.0501743046)
    p = p * ax + jnp.float32(0.0889789874)
    p = p * ax + jnp.float32(-0.2145988016)
    p = p * ax + jnp.float32(1.5707963050)
    a_pos = jnp.sqrt(jnp.maximum(1.0 - ax, 0.0)) * p
    return jnp.where(x >= 0, a_pos, jnp.float32(3.14159265358979) - a_pos)


def _eigen_ratio_block(S):
    k = jnp.float32(_K)
    mx = S[:, 0:1] / k
    my = S[:, 1:2] / k
    mz = S[:, 2:3] / k
    cxx = S[:, 3:4] / k - mx * mx
    cyy = S[:, 4:5] / k - my * my
    czz = S[:, 5:6] / k - mz * mz
    cxy = S[:, 6:7] / k - mx * my
    cxz = S[:, 7:8] / k - mx * mz
    cyz = S[:, 8:9] / k - my * mz

    q = (cxx + cyy + czz) * jnp.float32(1.0 / 3.0)
    p1 = cxy * cxy + cxz * cxz + cyz * cyz
    dxx = cxx - q
    dyy = cyy - q
    dzz = czz - q
    p2 = dxx * dxx + dyy * dyy + dzz * dzz + 2.0 * p1
    eps = jnp.float32(1e-30)
    safe = p2 > eps
    p = jnp.sqrt(jnp.maximum(p2, eps) * jnp.float32(1.0 / 6.0))
    inv_p = 1.0 / p
    b00 = dxx * inv_p
    b11 = dyy * inv_p
    b22 = dzz * inv_p
    b01 = cxy * inv_p
    b02 = cxz * inv_p
    b12 = cyz * inv_p
    detb = (b00 * (b11 * b22 - b12 * b12)
            - b01 * (b01 * b22 - b12 * b02)
            + b02 * (b01 * b12 - b11 * b02))
    r = jnp.clip(detb * 0.5, -1.0, 1.0)
    phi = _acos(r) * jnp.float32(1.0 / 3.0)
    e0 = q + 2.0 * p * jnp.cos(phi)
    e2 = q + 2.0 * p * jnp.cos(phi + jnp.float32(2.0943951023931953))
    e1 = 3.0 * q - e0 - e2
    return jnp.where(safe, e0 / e1, jnp.float32(1.0))


def _body(px_ref, pxT_ref, py_ref, pyT_ref, out_ref, er1_ref, er2_ref):
    b = pl.program_id(0)

    @pl.when(b == 0)
    def _():
        out_ref[:, :] = jnp.zeros((1, 1), jnp.float32)

    iota_i32 = jax.lax.broadcasted_iota(jnp.int32, (_R, _N), 1)

    def er_phase(p_ref, pT_ref, er_ref):
        pT = pT_ref[0]
        pts2 = jnp.sum(pT * pT, axis=0, keepdims=True)
        pT16 = pT.astype(jnp.bfloat16)

        def blk(i, _):
            r0 = i * _R
            a = p_ref[0, pl.ds(r0, _R), 0:3].astype(jnp.bfloat16)
            D = pts2 - 2.0 * jnp.dot(a, pT16, preferred_element_type=jnp.float32)
            M = _topk_mask(D)
            S = jnp.dot(M, p_ref[0], preferred_element_type=jnp.float32, precision=jax.lax.Precision.HIGHEST)
            er_ref[pl.ds(r0, _R), :] = _eigen_ratio_block(S)
            return 0

        jax.lax.fori_loop(0, _NBLK, blk, 0)

    er_phase(px_ref, pxT_ref, er1_ref)
    er_phase(py_ref, pyT_ref, er2_ref)

    def cross_phase(pa_ref, pbT_ref, era_ref, erb_ref):
        pbT = pbT_ref[0]
        pts2 = jnp.sum(pbT * pbT, axis=0, keepdims=True)
        pbT16 = pbT.astype(jnp.bfloat16)
        erb = erb_ref[:, :]

        def blk(i, sse):
            r0 = i * _R
            a = pa_ref[0, pl.ds(r0, _R), 0:3].astype(jnp.bfloat16)
            D = pts2 - 2.0 * jnp.dot(a, pbT16, preferred_element_type=jnp.float32)
            m = jnp.min(D, axis=1, keepdims=True)
            idxv = jnp.where(D == m, iota_i32, jnp.int32(2 * _N))
            amin = jnp.min(idxv, axis=1, keepdims=True)
            h = (iota_i32 == amin).astype(jnp.float32)
            corr = jnp.dot(h, erb, preferred_element_type=jnp.float32, precision=jax.lax.Precision.HIGHEST)
            d = era_ref[pl.ds(r0, _R), :] - corr
            return sse + jnp.sum(d * d, axis=(0, 1), keepdims=True)

        return jax.lax.fori_loop(0, _NBLK, blk, jnp.zeros((1, 1), jnp.float32))

    sse_x = cross_phase(px_ref, pyT_ref, er1_ref, er2_ref)
    sse_y = cross_phase(py_ref, pxT_ref, er2_ref, er1_ref)
    out_ref[:, :] += sse_x + sse_y


def _moments(pts):
    x = pts[..., 0:1]
    y = pts[..., 1:2]
    z = pts[..., 2:3]
    zeros = jnp.zeros(pts.shape[:-1] + (7,), pts.dtype)
    return jnp.concatenate(
        [x, y, z, x * x, y * y, z * z, x * y, x * z, y * z, zeros], axis=-1)


@jax.jit
def kernel(x, y):
    x3 = x[..., :3].astype(jnp.float32)
    y3 = y[..., :3].astype(jnp.float32)
    px = _moments(x3)
    py = _moments(y3)
    pxT = jnp.swapaxes(x3, 1, 2)
    pyT = jnp.swapaxes(y3, 1, 2)

    bspec_p = pl.BlockSpec((1, _N, 16), lambda b: (b, 0, 0))
    bspec_t = pl.BlockSpec((1, 3, _N), lambda b: (b, 0, 0))
    acc = pl.pallas_call(
        _body,
        grid=(x.shape[0],),
        in_specs=[bspec_p, bspec_t, bspec_p, bspec_t],
        out_specs=pl.BlockSpec((1, 1), lambda b: (0, 0)),
        out_shape=jax.ShapeDtypeStruct((1, 1), jnp.float32),
        scratch_shapes=[pltpu.VMEM((_N, 1), jnp.float32),
                        pltpu.VMEM((_N, 1), jnp.float32)],
    )(px, pxT, py, pyT)
    return acc[0, 0] * jnp.float32(0.5 / (_N * x.shape[0]))


# speedup vs baseline: 2.5936x; 1.0292x over previous
import functools

import jax
import jax.numpy as jnp
from jax.experimental import pallas as pl
from jax.experimental.pallas import tpu as pltpu

_K = 16
_N = 4096
_R = 128
_NBLK = _N // _R


def _topk_mask(D):
    big = jnp.float32(1e30)
    for _ in range(_K):
        m = jnp.min(D, axis=1, keepdims=True)
        D = jnp.where(D == m, big, D)
    return (D >= jnp.float32(1e29)).astype(jnp.float32)


def _acos(x):
    ax = jnp.abs(x)
    p = jnp.float32(-0.0012624911)
    p = p * ax + jnp.float32(0.0066700901)
    p = p * ax + jnp.float32(-0.0170881256)
    p = p * ax + jnp.float32(0.0308918810)
    p = p * ax + jnp.float32(-0.0501743046)
    p = p * ax + jnp.float32(0.0889789874)
    p = p * ax + jnp.float32(-0.2145988016)
    p = p * ax + jnp.float32(1.5707963050)
    a_pos = jnp.sqrt(jnp.maximum(1.0 - ax, 0.0)) * p
    return jnp.where(x >= 0, a_pos, jnp.float32(3.14159265358979) - a_pos)


def _eigen_ratio_block(S):
    k = jnp.float32(_K)
    mx = S[:, 0:1] / k
    my = S[:, 1:2] / k
    mz = S[:, 2:3] / k
    cxx = S[:, 3:4] / k - mx * mx
    cyy = S[:, 4:5] / k - my * my
    czz = S[:, 5:6] / k - mz * mz
    cxy = S[:, 6:7] / k - mx * my
    cxz = S[:, 7:8] / k - mx * mz
    cyz = S[:, 8:9] / k - my * mz

    q = (cxx + cyy + czz) * jnp.float32(1.0 / 3.0)
    p1 = cxy * cxy + cxz * cxz + cyz * cyz
    dxx = cxx - q
    dyy = cyy - q
    dzz = czz - q
    p2 = dxx * dxx + dyy * dyy + dzz * dzz + 2.0 * p1
    eps = jnp.float32(1e-30)
    safe = p2 > eps
    p = jnp.sqrt(jnp.maximum(p2, eps) * jnp.float32(1.0 / 6.0))
    inv_p = 1.0 / p
    b00 = dxx * inv_p
    b11 = dyy * inv_p
    b22 = dzz * inv_p
    b01 = cxy * inv_p
    b02 = cxz * inv_p
    b12 = cyz * inv_p
    detb = (b00 * (b11 * b22 - b12 * b12)
            - b01 * (b01 * b22 - b12 * b02)
            + b02 * (b01 * b12 - b11 * b02))
    r = jnp.clip(detb * 0.5, -1.0, 1.0)
    phi = _acos(r) * jnp.float32(1.0 / 3.0)
    e0 = q + 2.0 * p * jnp.cos(phi)
    e2 = q + 2.0 * p * jnp.cos(phi + jnp.float32(2.0943951023931953))
    e1 = 3.0 * q - e0 - e2
    return jnp.where(safe, e0 / e1, jnp.float32(1.0))


def _body(px_ref, pxT_ref, py_ref, pyT_ref, out_ref, er1_ref, er2_ref):
    b = pl.program_id(0)

    @pl.when(b == 0)
    def _():
        out_ref[:, :] = jnp.zeros((1, 1), jnp.float32)

    def er_phase(p_ref, pT_ref, er_ref):
        pT = pT_ref[0]
        pts2 = jnp.sum(pT * pT, axis=0, keepdims=True)
        pT16 = pT.astype(jnp.bfloat16)

        def blk(i, _):
            r0 = i * _R
            a = p_ref[0, pl.ds(r0, _R), 0:3].astype(jnp.bfloat16)
            D = pts2 - 2.0 * jnp.dot(a, pT16, preferred_element_type=jnp.float32)
            M = _topk_mask(D)
            S = jnp.dot(M, p_ref[0], preferred_element_type=jnp.float32, precision=jax.lax.Precision.HIGHEST)
            er = _eigen_ratio_block(S)
            er_ref[pl.ds(r0, _R), :] = jnp.concatenate(
                [er, jnp.ones((_R, 1), jnp.float32)], axis=1)
            return 0

        jax.lax.fori_loop(0, _NBLK, blk, 0)

    er_phase(px_ref, pxT_ref, er1_ref)
    er_phase(py_ref, pyT_ref, er2_ref)

    def cross_phase(pa_ref, pbT_ref, era_ref, erb_ref):
        pbT = pbT_ref[0]
        pts2 = jnp.sum(pbT * pbT, axis=0, keepdims=True)
        pbT16 = pbT.astype(jnp.bfloat16)
        erb = erb_ref[:, :]

        def blk(i, sse):
            r0 = i * _R
            a = pa_ref[0, pl.ds(r0, _R), 0:3].astype(jnp.bfloat16)
            D = pts2 - 2.0 * jnp.dot(a, pbT16, preferred_element_type=jnp.float32)
            m = jnp.min(D, axis=1, keepdims=True)
            hb = (D == m).astype(jnp.float32)
            c2 = jnp.dot(hb, erb, preferred_element_type=jnp.float32,
                         precision=jax.lax.Precision.HIGHEST)
            corr = c2[:, 0:1] / c2[:, 1:2]
            d = era_ref[pl.ds(r0, _R), 0:1] - corr
            return sse + jnp.sum(d * d, axis=(0, 1), keepdims=True)

        return jax.lax.fori_loop(0, _NBLK, blk, jnp.zeros((1, 1), jnp.float32))

    sse_x = cross_phase(px_ref, pyT_ref, er1_ref, er2_ref)
    sse_y = cross_phase(py_ref, pxT_ref, er2_ref, er1_ref)
    out_ref[:, :] += sse_x + sse_y


def _moments(pts):
    x = pts[..., 0:1]
    y = pts[..., 1:2]
    z = pts[..., 2:3]
    zeros = jnp.zeros(pts.shape[:-1] + (7,), pts.dtype)
    return jnp.concatenate(
        [x, y, z, x * x, y * y, z * z, x * y, x * z, y * z, zeros], axis=-1)


@jax.jit
def kernel(x, y):
    x3 = x[..., :3].astype(jnp.float32)
    y3 = y[..., :3].astype(jnp.float32)
    px = _moments(x3)
    py = _moments(y3)
    pxT = jnp.swapaxes(x3, 1, 2)
    pyT = jnp.swapaxes(y3, 1, 2)

    bspec_p = pl.BlockSpec((1, _N, 16), lambda b: (b, 0, 0))
    bspec_t = pl.BlockSpec((1, 3, _N), lambda b: (b, 0, 0))
    acc = pl.pallas_call(
        _body,
        grid=(x.shape[0],),
        in_specs=[bspec_p, bspec_t, bspec_p, bspec_t],
        out_specs=pl.BlockSpec((1, 1), lambda b: (0, 0)),
        out_shape=jax.ShapeDtypeStruct((1, 1), jnp.float32),
        scratch_shapes=[pltpu.VMEM((_N, 2), jnp.float32),
                        pltpu.VMEM((_N, 2), jnp.float32)],
    )(px, pxT, py, pyT)
    return acc[0, 0] * jnp.float32(0.5 / (_N * x.shape[0]))


# speedup vs baseline: 2.9846x; 1.1507x over previous
import functools

import jax
import jax.numpy as jnp
from jax.experimental import pallas as pl
from jax.experimental.pallas import tpu as pltpu
from jax.experimental.pallas import tpu_sc as plsc

_K = 16
_N = 4096
_R = 128
_NBLK = _N // _R
_NW = 32
_SEG = 4 * _N // _NW


def _topk_mask(D):
    big = jnp.float32(1e30)
    for _ in range(_K):
        m = jnp.min(D, axis=1, keepdims=True)
        D = jnp.where(D == m, big, D)
    return (D >= jnp.float32(1e29)).astype(jnp.float32)


def _acos(x):
    ax = jnp.abs(x)
    p = jnp.float32(-0.0012624911)
    p = p * ax + jnp.float32(0.0066700901)
    p = p * ax + jnp.float32(-0.0170881256)
    p = p * ax + jnp.float32(0.0308918810)
    p = p * ax + jnp.float32(-0.0501743046)
    p = p * ax + jnp.float32(0.0889789874)
    p = p * ax + jnp.float32(-0.2145988016)
    p = p * ax + jnp.float32(1.5707963050)
    a_pos = jnp.sqrt(jnp.maximum(1.0 - ax, 0.0)) * p
    return jnp.where(x >= 0, a_pos, jnp.float32(3.14159265358979) - a_pos)


def _eigen_ratio_block(S):
    k = jnp.float32(_K)
    mx = S[:, 0:1] / k
    my = S[:, 1:2] / k
    mz = S[:, 2:3] / k
    cxx = S[:, 3:4] / k - mx * mx
    cyy = S[:, 4:5] / k - my * my
    czz = S[:, 5:6] / k - mz * mz
    cxy = S[:, 6:7] / k - mx * my
    cxz = S[:, 7:8] / k - mx * mz
    cyz = S[:, 8:9] / k - my * mz

    q = (cxx + cyy + czz) * jnp.float32(1.0 / 3.0)
    p1 = cxy * cxy + cxz * cxz + cyz * cyz
    dxx = cxx - q
    dyy = cyy - q
    dzz = czz - q
    p2 = dxx * dxx + dyy * dyy + dzz * dzz + 2.0 * p1
    eps = jnp.float32(1e-30)
    safe = p2 > eps
    p = jnp.sqrt(jnp.maximum(p2, eps) * jnp.float32(1.0 / 6.0))
    inv_p = 1.0 / p
    b00 = dxx * inv_p
    b11 = dyy * inv_p
    b22 = dzz * inv_p
    b01 = cxy * inv_p
    b02 = cxz * inv_p
    b12 = cyz * inv_p
    detb = (b00 * (b11 * b22 - b12 * b12)
            - b01 * (b01 * b22 - b12 * b02)
            + b02 * (b01 * b12 - b11 * b02))
    r = jnp.clip(detb * 0.5, -1.0, 1.0)
    phi = _acos(r) * jnp.float32(1.0 / 3.0)
    e0 = q + 2.0 * p * jnp.cos(phi)
    e2 = q + 2.0 * p * jnp.cos(phi + jnp.float32(2.0943951023931953))
    e1 = 3.0 * q - e0 - e2
    return jnp.where(safe, e0 / e1, jnp.float32(1.0))


def _tc_body(px_ref, pxT_ref, py_ref, pyT_ref,
             er1_ref, er2_ref, idx1_ref, idx2_ref):
    iota_i = jax.lax.broadcasted_iota(jnp.int32, (_R, _N), 1)

    def er_phase(p_ref, pT_ref, er_ref):
        pT = pT_ref[0]
        pts2 = jnp.sum(pT * pT, axis=0, keepdims=True)
        pT16 = pT.astype(jnp.bfloat16)

        def blk(i, _):
            r0 = i * _R
            a = p_ref[0, pl.ds(r0, _R), 0:3].astype(jnp.bfloat16)
            D = pts2 - 2.0 * jnp.dot(a, pT16, preferred_element_type=jnp.float32)
            M = _topk_mask(D)
            S = jnp.dot(M, p_ref[0], preferred_element_type=jnp.float32,
                        precision=jax.lax.Precision.HIGHEST)
            er_ref[0, pl.ds(r0, _R), :] = _eigen_ratio_block(S)
            return 0

        jax.lax.fori_loop(0, _NBLK, blk, 0)

    er_phase(px_ref, pxT_ref, er1_ref)
    er_phase(py_ref, pyT_ref, er2_ref)

    def cross_phase(pa_ref, pbT_ref, idx_ref):
        pbT = pbT_ref[0]
        pts2 = jnp.sum(pbT * pbT, axis=0, keepdims=True)
        pbT16 = pbT.astype(jnp.bfloat16)

        def blk(i, _):
            r0 = i * _R
            a = pa_ref[0, pl.ds(r0, _R), 0:3].astype(jnp.bfloat16)
            D = pts2 - 2.0 * jnp.dot(a, pbT16, preferred_element_type=jnp.float32)
            m = jnp.min(D, axis=1, keepdims=True)
            key = jnp.where(D == m, iota_i, jnp.int32(2 * _N))
            amin = jnp.min(key, axis=1, keepdims=True)
            idx_ref[0, pl.ds(r0, _R), :] = amin
            return 0

        jax.lax.fori_loop(0, _NBLK, blk, 0)

    cross_phase(px_ref, pyT_ref, idx1_ref)
    cross_phase(py_ref, pxT_ref, idx2_ref)


def _sc_body(table_ref, idx_ref, era_ref, out_ref,
             idx_v, era_v, val_v, acc_v, sem):
    c = jax.lax.axis_index("c")
    s = jax.lax.axis_index("s")
    wid = s * 2 + c
    pltpu.sync_copy(idx_ref.at[wid], idx_v)
    pltpu.sync_copy(era_ref.at[wid], era_v)
    for j in range(4):
        pltpu.async_copy(table_ref.at[idx_v.at[j]],
                         val_v.at[pl.ds(j * 128, 128)], sem).wait()
    acc = jnp.zeros((16,), jnp.float32)
    for j in range(_SEG // 16):
        d = era_v[pl.ds(j * 16, 16)] - val_v[pl.ds(j * 16, 16)]
        acc = acc + d * d
    acc_v[...] = acc
    pltpu.sync_copy(acc_v, out_ref.at[wid])


def _moments(pts):
    x = pts[..., 0:1]
    y = pts[..., 1:2]
    z = pts[..., 2:3]
    zeros = jnp.zeros(pts.shape[:-1] + (7,), pts.dtype)
    return jnp.concatenate(
        [x, y, z, x * x, y * y, z * z, x * y, x * z, y * z, zeros], axis=-1)


@jax.jit
def kernel(x, y):
    B = x.shape[0]
    x3 = x[..., :3].astype(jnp.float32)
    y3 = y[..., :3].astype(jnp.float32)
    px = _moments(x3)
    py = _moments(y3)
    pxT = jnp.swapaxes(x3, 1, 2)
    pyT = jnp.swapaxes(y3, 1, 2)

    bspec_p = pl.BlockSpec((1, _N, 16), lambda b: (b, 0, 0))
    bspec_t = pl.BlockSpec((1, 3, _N), lambda b: (b, 0, 0))
    bspec_o = pl.BlockSpec((1, _N, 1), lambda b: (b, 0, 0))
    f32 = jnp.float32
    er1, er2, idx1, idx2 = pl.pallas_call(
        _tc_body,
        grid=(B,),
        in_specs=[bspec_p, bspec_t, bspec_p, bspec_t],
        out_specs=[bspec_o, bspec_o, bspec_o, bspec_o],
        out_shape=[jax.ShapeDtypeStruct((B, _N, 1), f32),
                   jax.ShapeDtypeStruct((B, _N, 1), f32),
                   jax.ShapeDtypeStruct((B, _N, 1), jnp.int32),
                   jax.ShapeDtypeStruct((B, _N, 1), jnp.int32)],
    )(px, pxT, py, pyT)

    er1f = er1[..., 0]
    er2f = er2[..., 0]
    table = jnp.concatenate([er2f.reshape(-1), er1f.reshape(-1)])
    offs = (_N * jnp.arange(B, dtype=jnp.int32))[:, None]
    idx_all = jnp.concatenate([(idx1[..., 0] + offs).reshape(-1),
                               (idx2[..., 0] + offs + B * _N).reshape(-1)])
    era_all = jnp.concatenate([er1f.reshape(-1), er2f.reshape(-1)])

    sck = pl.kernel(
        _sc_body,
        out_type=jax.ShapeDtypeStruct((_NW, 16), f32),
        mesh=plsc.VectorSubcoreMesh(core_axis_name="c", subcore_axis_name="s"),
        scratch_types=[pltpu.VMEM((4, 128), jnp.int32),
                       pltpu.VMEM((_SEG,), f32),
                       pltpu.VMEM((_SEG,), f32),
                       pltpu.VMEM((16,), f32),
                       pltpu.SemaphoreType.DMA],
    )
    partials = sck(table,
                   idx_all.reshape(_NW, 4, 128),
                   era_all.reshape(_NW, _SEG))
    return jnp.sum(partials) * jnp.float32(0.5 / (_N * B))


# speedup vs baseline: 3.0029x; 1.0061x over previous
import functools

import jax
import jax.numpy as jnp
from jax.experimental import pallas as pl
from jax.experimental.pallas import tpu as pltpu
from jax.experimental.pallas import tpu_sc as plsc

_K = 16
_N = 4096
_R = 256
_NBLK = _N // _R
_NW = 32
_SEG = 4 * _N // _NW


def _topk_mask(D):
    big = jnp.float32(1e30)
    for _ in range(_K):
        m = jnp.min(D, axis=1, keepdims=True)
        D = jnp.where(D == m, big, D)
    return (D >= jnp.float32(1e29)).astype(jnp.float32)


def _acos(x):
    ax = jnp.abs(x)
    p = jnp.float32(-0.0012624911)
    p = p * ax + jnp.float32(0.0066700901)
    p = p * ax + jnp.float32(-0.0170881256)
    p = p * ax + jnp.float32(0.0308918810)
    p = p * ax + jnp.float32(-0.0501743046)
    p = p * ax + jnp.float32(0.0889789874)
    p = p * ax + jnp.float32(-0.2145988016)
    p = p * ax + jnp.float32(1.5707963050)
    a_pos = jnp.sqrt(jnp.maximum(1.0 - ax, 0.0)) * p
    return jnp.where(x >= 0, a_pos, jnp.float32(3.14159265358979) - a_pos)


def _eigen_ratio_block(S):
    k = jnp.float32(_K)
    mx = S[:, 0:1] / k
    my = S[:, 1:2] / k
    mz = S[:, 2:3] / k
    cxx = S[:, 3:4] / k - mx * mx
    cyy = S[:, 4:5] / k - my * my
    czz = S[:, 5:6] / k - mz * mz
    cxy = S[:, 6:7] / k - mx * my
    cxz = S[:, 7:8] / k - mx * mz
    cyz = S[:, 8:9] / k - my * mz

    q = (cxx + cyy + czz) * jnp.float32(1.0 / 3.0)
    p1 = cxy * cxy + cxz * cxz + cyz * cyz
    dxx = cxx - q
    dyy = cyy - q
    dzz = czz - q
    p2 = dxx * dxx + dyy * dyy + dzz * dzz + 2.0 * p1
    eps = jnp.float32(1e-30)
    safe = p2 > eps
    p = jnp.sqrt(jnp.maximum(p2, eps) * jnp.float32(1.0 / 6.0))
    inv_p = 1.0 / p
    b00 = dxx * inv_p
    b11 = dyy * inv_p
    b22 = dzz * inv_p
    b01 = cxy * inv_p
    b02 = cxz * inv_p
    b12 = cyz * inv_p
    detb = (b00 * (b11 * b22 - b12 * b12)
            - b01 * (b01 * b22 - b12 * b02)
            + b02 * (b01 * b12 - b11 * b02))
    r = jnp.clip(detb * 0.5, -1.0, 1.0)
    phi = _acos(r) * jnp.float32(1.0 / 3.0)
    e0 = q + 2.0 * p * jnp.cos(phi)
    e2 = q + 2.0 * p * jnp.cos(phi + jnp.float32(2.0943951023931953))
    e1 = 3.0 * q - e0 - e2
    return jnp.where(safe, e0 / e1, jnp.float32(1.0))


def _tc_body(px_ref, pxT_ref, py_ref, pyT_ref,
             er1_ref, er2_ref, idx1_ref, idx2_ref):
    iota_i = jax.lax.broadcasted_iota(jnp.int32, (_R, _N), 1)

    def er_phase(p_ref, pT_ref, er_ref):
        pT = pT_ref[0]
        pts2 = jnp.sum(pT * pT, axis=0, keepdims=True)
        pT16 = pT.astype(jnp.bfloat16)

        def blk(i, _):
            r0 = i * _R
            a = p_ref[0, pl.ds(r0, _R), 0:3].astype(jnp.bfloat16)
            D = pts2 - 2.0 * jnp.dot(a, pT16, preferred_element_type=jnp.float32)
            M = _topk_mask(D)
            S = jnp.dot(M, p_ref[0], preferred_element_type=jnp.float32,
                        precision=jax.lax.Precision.HIGHEST)
            er_ref[0, pl.ds(r0, _R), :] = _eigen_ratio_block(S)
            return 0

        jax.lax.fori_loop(0, _NBLK, blk, 0)

    er_phase(px_ref, pxT_ref, er1_ref)
    er_phase(py_ref, pyT_ref, er2_ref)

    def cross_phase(pa_ref, pbT_ref, idx_ref):
        pbT = pbT_ref[0]
        pts2 = jnp.sum(pbT * pbT, axis=0, keepdims=True)
        pbT16 = pbT.astype(jnp.bfloat16)

        def blk(i, _):
            r0 = i * _R
            a = pa_ref[0, pl.ds(r0, _R), 0:3].astype(jnp.bfloat16)
            D = pts2 - 2.0 * jnp.dot(a, pbT16, preferred_element_type=jnp.float32)
            m = jnp.min(D, axis=1, keepdims=True)
            key = jnp.where(D == m, iota_i, jnp.int32(2 * _N))
            amin = jnp.min(key, axis=1, keepdims=True)
            idx_ref[0, pl.ds(r0, _R), :] = amin
            return 0

        jax.lax.fori_loop(0, _NBLK, blk, 0)

    cross_phase(px_ref, pyT_ref, idx1_ref)
    cross_phase(py_ref, pxT_ref, idx2_ref)


def _sc_body(table_ref, idx_ref, era_ref, out_ref,
             idx_v, era_v, val_v, acc_v, sem):
    c = jax.lax.axis_index("c")
    s = jax.lax.axis_index("s")
    wid = s * 2 + c
    pltpu.sync_copy(idx_ref.at[wid], idx_v)
    pltpu.sync_copy(era_ref.at[wid], era_v)
    for j in range(4):
        pltpu.async_copy(table_ref.at[idx_v.at[j]],
                         val_v.at[pl.ds(j * 128, 128)], sem).wait()
    acc = jnp.zeros((16,), jnp.float32)
    for j in range(_SEG // 16):
        d = era_v[pl.ds(j * 16, 16)] - val_v[pl.ds(j * 16, 16)]
        acc = acc + d * d
    acc_v[...] = acc
    pltpu.sync_copy(acc_v, out_ref.at[wid])


def _moments(pts):
    x = pts[..., 0:1]
    y = pts[..., 1:2]
    z = pts[..., 2:3]
    zeros = jnp.zeros(pts.shape[:-1] + (7,), pts.dtype)
    return jnp.concatenate(
        [x, y, z, x * x, y * y, z * z, x * y, x * z, y * z, zeros], axis=-1)


@jax.jit
def kernel(x, y):
    B = x.shape[0]
    x3 = x[..., :3].astype(jnp.float32)
    y3 = y[..., :3].astype(jnp.float32)
    px = _moments(x3)
    py = _moments(y3)
    pxT = jnp.swapaxes(x3, 1, 2)
    pyT = jnp.swapaxes(y3, 1, 2)

    bspec_p = pl.BlockSpec((1, _N, 16), lambda b: (b, 0, 0))
    bspec_t = pl.BlockSpec((1, 3, _N), lambda b: (b, 0, 0))
    bspec_o = pl.BlockSpec((1, _N, 1), lambda b: (b, 0, 0))
    f32 = jnp.float32
    er1, er2, idx1, idx2 = pl.pallas_call(
        _tc_body,
        grid=(B,),
        in_specs=[bspec_p, bspec_t, bspec_p, bspec_t],
        out_specs=[bspec_o, bspec_o, bspec_o, bspec_o],
        out_shape=[jax.ShapeDtypeStruct((B, _N, 1), f32),
                   jax.ShapeDtypeStruct((B, _N, 1), f32),
                   jax.ShapeDtypeStruct((B, _N, 1), jnp.int32),
                   jax.ShapeDtypeStruct((B, _N, 1), jnp.int32)],
    )(px, pxT, py, pyT)

    er1f = er1[..., 0]
    er2f = er2[..., 0]
    table = jnp.concatenate([er2f.reshape(-1), er1f.reshape(-1)])
    offs = (_N * jnp.arange(B, dtype=jnp.int32))[:, None]
    idx_all = jnp.concatenate([(idx1[..., 0] + offs).reshape(-1),
                               (idx2[..., 0] + offs + B * _N).reshape(-1)])
    era_all = jnp.concatenate([er1f.reshape(-1), er2f.reshape(-1)])

    sck = pl.kernel(
        _sc_body,
        out_type=jax.ShapeDtypeStruct((_NW, 16), f32),
        mesh=plsc.VectorSubcoreMesh(core_axis_name="c", subcore_axis_name="s"),
        scratch_types=[pltpu.VMEM((4, 128), jnp.int32),
                       pltpu.VMEM((_SEG,), f32),
                       pltpu.VMEM((_SEG,), f32),
                       pltpu.VMEM((16,), f32),
                       pltpu.SemaphoreType.DMA],
    )
    partials = sck(table,
                   idx_all.reshape(_NW, 4, 128),
                   era_all.reshape(_NW, _SEG))
    return jnp.sum(partials) * jnp.float32(0.5 / (_N * B))


# speedup vs baseline: 3.0036x; 1.0002x over previous
import functools

import jax
import jax.numpy as jnp
from jax.experimental import pallas as pl
from jax.experimental.pallas import tpu as pltpu
from jax.experimental.pallas import tpu_sc as plsc

_K = 16
_N = 4096
_R = 256
_NBLK = _N // _R
_NW = 32
_SEG = 4 * _N // _NW


def _topk_mask(D):
    big = jnp.float32(1e30)
    for _ in range(_K):
        m = jnp.min(D, axis=1, keepdims=True)
        D = jnp.where(D == m, big, D)
    return (D >= jnp.float32(1e29)).astype(jnp.float32)


def _acos(x):
    ax = jnp.abs(x)
    p = jnp.float32(-0.0012624911)
    p = p * ax + jnp.float32(0.0066700901)
    p = p * ax + jnp.float32(-0.0170881256)
    p = p * ax + jnp.float32(0.0308918810)
    p = p * ax + jnp.float32(-0.0501743046)
    p = p * ax + jnp.float32(0.0889789874)
    p = p * ax + jnp.float32(-0.2145988016)
    p = p * ax + jnp.float32(1.5707963050)
    a_pos = jnp.sqrt(jnp.maximum(1.0 - ax, 0.0)) * p
    return jnp.where(x >= 0, a_pos, jnp.float32(3.14159265358979) - a_pos)


def _eigen_ratio_block(S):
    k = jnp.float32(_K)
    mx = S[:, 0:1] / k
    my = S[:, 1:2] / k
    mz = S[:, 2:3] / k
    cxx = S[:, 3:4] / k - mx * mx
    cyy = S[:, 4:5] / k - my * my
    czz = S[:, 5:6] / k - mz * mz
    cxy = S[:, 6:7] / k - mx * my
    cxz = S[:, 7:8] / k - mx * mz
    cyz = S[:, 8:9] / k - my * mz

    q = (cxx + cyy + czz) * jnp.float32(1.0 / 3.0)
    p1 = cxy * cxy + cxz * cxz + cyz * cyz
    dxx = cxx - q
    dyy = cyy - q
    dzz = czz - q
    p2 = dxx * dxx + dyy * dyy + dzz * dzz + 2.0 * p1
    eps = jnp.float32(1e-30)
    safe = p2 > eps
    p = jnp.sqrt(jnp.maximum(p2, eps) * jnp.float32(1.0 / 6.0))
    inv_p = 1.0 / p
    b00 = dxx * inv_p
    b11 = dyy * inv_p
    b22 = dzz * inv_p
    b01 = cxy * inv_p
    b02 = cxz * inv_p
    b12 = cyz * inv_p
    detb = (b00 * (b11 * b22 - b12 * b12)
            - b01 * (b01 * b22 - b12 * b02)
            + b02 * (b01 * b12 - b11 * b02))
    r = jnp.clip(detb * 0.5, -1.0, 1.0)
    phi = _acos(r) * jnp.float32(1.0 / 3.0)
    e0 = q + 2.0 * p * jnp.cos(phi)
    e2 = q + 2.0 * p * jnp.cos(phi + jnp.float32(2.0943951023931953))
    e1 = 3.0 * q - e0 - e2
    return jnp.where(safe, e0 / e1, jnp.float32(1.0))


def _tc_body(px_ref, pxT_ref, py_ref, pyT_ref,
             er1_ref, er2_ref, idx1_ref, idx2_ref):
    iota_i = jax.lax.broadcasted_iota(jnp.int32, (_R, _N), 1)

    def er_phase(p_ref, pT_ref, er_ref):
        pT = pT_ref[0]
        pts2 = jnp.sum(pT * pT, axis=0, keepdims=True)
        pT16 = pT.astype(jnp.bfloat16)

        def blk(i, _):
            r0 = i * _R
            a = p_ref[0, pl.ds(r0, _R), 0:3].astype(jnp.bfloat16)
            D = pts2 - 2.0 * jnp.dot(a, pT16, preferred_element_type=jnp.float32)
            M = _topk_mask(D)
            S = jnp.dot(M, p_ref[0], preferred_element_type=jnp.float32,
                        precision=jax.lax.Precision.HIGHEST)
            er_ref[0, pl.ds(r0, _R), :] = _eigen_ratio_block(S)
            return 0

        jax.lax.fori_loop(0, _NBLK, blk, 0)

    er_phase(px_ref, pxT_ref, er1_ref)
    er_phase(py_ref, pyT_ref, er2_ref)

    def cross_phase(pa_ref, pbT_ref, idx_ref):
        pbT = pbT_ref[0]
        pts2 = jnp.sum(pbT * pbT, axis=0, keepdims=True)
        pbT16 = pbT.astype(jnp.bfloat16)

        def blk(i, _):
            r0 = i * _R
            a = pa_ref[0, pl.ds(r0, _R), 0:3].astype(jnp.bfloat16)
            D = pts2 - 2.0 * jnp.dot(a, pbT16, preferred_element_type=jnp.float32)
            m = jnp.min(D, axis=1, keepdims=True)
            key = jnp.where(D == m, iota_i, jnp.int32(2 * _N))
            amin = jnp.min(key, axis=1, keepdims=True)
            idx_ref[0, pl.ds(r0, _R), :] = amin
            return 0

        jax.lax.fori_loop(0, _NBLK, blk, 0)

    cross_phase(px_ref, pyT_ref, idx1_ref)
    cross_phase(py_ref, pxT_ref, idx2_ref)


def _sc_body(table_ref, idx_ref, era_ref, out_ref,
             idx_v, era_v, val_v, acc_v, sem):
    c = jax.lax.axis_index("c")
    s = jax.lax.axis_index("s")
    wid = s * 2 + c
    pltpu.sync_copy(idx_ref.at[wid], idx_v)
    pltpu.sync_copy(era_ref.at[wid], era_v)
    for j in range(4):
        pltpu.async_copy(table_ref.at[idx_v.at[j]],
                         val_v.at[pl.ds(j * 128, 128)], sem).wait()
    acc = jnp.zeros((16,), jnp.float32)
    for j in range(_SEG // 16):
        d = era_v[pl.ds(j * 16, 16)] - val_v[pl.ds(j * 16, 16)]
        acc = acc + d * d
    acc_v[...] = acc
    pltpu.sync_copy(acc_v, out_ref.at[wid])


def _moments(pts):
    x = pts[..., 0:1]
    y = pts[..., 1:2]
    z = pts[..., 2:3]
    zeros = jnp.zeros(pts.shape[:-1] + (7,), pts.dtype)
    return jnp.concatenate(
        [x, y, z, x * x, y * y, z * z, x * y, x * z, y * z, zeros], axis=-1)


@jax.jit
def kernel(x, y):
    B = x.shape[0]
    x3 = x[..., :3].astype(jnp.float32)
    y3 = y[..., :3].astype(jnp.float32)
    px = _moments(x3)
    py = _moments(y3)
    pxT = jnp.swapaxes(x3, 1, 2)
    pyT = jnp.swapaxes(y3, 1, 2)

    bspec_p = pl.BlockSpec((1, _N, 16), lambda b: (b, 0, 0))
    bspec_t = pl.BlockSpec((1, 3, _N), lambda b: (b, 0, 0))
    bspec_o = pl.BlockSpec((1, _N, 1), lambda b: (b, 0, 0))
    f32 = jnp.float32
    er1, er2, idx1, idx2 = pl.pallas_call(
        _tc_body,
        grid=(B,),
        in_specs=[bspec_p, bspec_t, bspec_p, bspec_t],
        out_specs=[bspec_o, bspec_o, bspec_o, bspec_o],
        out_shape=[jax.ShapeDtypeStruct((B, _N, 1), f32),
                   jax.ShapeDtypeStruct((B, _N, 1), f32),
                   jax.ShapeDtypeStruct((B, _N, 1), jnp.int32),
                   jax.ShapeDtypeStruct((B, _N, 1), jnp.int32)],
        compiler_params=pltpu.CompilerParams(
            dimension_semantics=("parallel",)),
    )(px, pxT, py, pyT)

    er1f = er1[..., 0]
    er2f = er2[..., 0]
    table = jnp.concatenate([er2f.reshape(-1), er1f.reshape(-1)])
    offs = (_N * jnp.arange(B, dtype=jnp.int32))[:, None]
    idx_all = jnp.concatenate([(idx1[..., 0] + offs).reshape(-1),
                               (idx2[..., 0] + offs + B * _N).reshape(-1)])
    era_all = jnp.concatenate([er1f.reshape(-1), er2f.reshape(-1)])

    sck = pl.kernel(
        _sc_body,
        out_type=jax.ShapeDtypeStruct((_NW, 16), f32),
        mesh=plsc.VectorSubcoreMesh(core_axis_name="c", subcore_axis_name="s"),
        scratch_types=[pltpu.VMEM((4, 128), jnp.int32),
                       pltpu.VMEM((_SEG,), f32),
                       pltpu.VMEM((_SEG,), f32),
                       pltpu.VMEM((16,), f32),
                       pltpu.SemaphoreType.DMA],
    )
    partials = sck(table,
                   idx_all.reshape(_NW, 4, 128),
                   era_all.reshape(_NW, _SEG))
    return jnp.sum(partials) * jnp.float32(0.5 / (_N * B))


# speedup vs baseline: 3.5004x; 1.1654x over previous
import functools

import jax
import jax.numpy as jnp
from jax.experimental import pallas as pl
from jax.experimental.pallas import tpu as pltpu
from jax.experimental.pallas import tpu_sc as plsc

_K = 16
_N = 4096
_R = 256
_NBLK = _N // _R
_NW = 32
_SEG = 4 * _N // _NW


def _topk_mask(D):
    big = jnp.float32(1e30)
    for _ in range(_K):
        m = jnp.min(D, axis=1, keepdims=True)
        D = jnp.where(D == m, big, D)
    return (D >= jnp.float32(1e29)).astype(jnp.float32)


def _acos(x):
    ax = jnp.abs(x)
    p = jnp.float32(-0.0012624911)
    p = p * ax + jnp.float32(0.0066700901)
    p = p * ax + jnp.float32(-0.0170881256)
    p = p * ax + jnp.float32(0.0308918810)
    p = p * ax + jnp.float32(-0.0501743046)
    p = p * ax + jnp.float32(0.0889789874)
    p = p * ax + jnp.float32(-0.2145988016)
    p = p * ax + jnp.float32(1.5707963050)
    a_pos = jnp.sqrt(jnp.maximum(1.0 - ax, 0.0)) * p
    return jnp.where(x >= 0, a_pos, jnp.float32(3.14159265358979) - a_pos)


def _eigen_ratio_block(S):
    k = jnp.float32(_K)
    mx = S[:, 0:1] / k
    my = S[:, 1:2] / k
    mz = S[:, 2:3] / k
    cxx = S[:, 3:4] / k - mx * mx
    cyy = S[:, 4:5] / k - my * my
    czz = S[:, 5:6] / k - mz * mz
    cxy = S[:, 6:7] / k - mx * my
    cxz = S[:, 7:8] / k - mx * mz
    cyz = S[:, 8:9] / k - my * mz

    q = (cxx + cyy + czz) * jnp.float32(1.0 / 3.0)
    p1 = cxy * cxy + cxz * cxz + cyz * cyz
    dxx = cxx - q
    dyy = cyy - q
    dzz = czz - q
    p2 = dxx * dxx + dyy * dyy + dzz * dzz + 2.0 * p1
    eps = jnp.float32(1e-30)
    safe = p2 > eps
    p = jnp.sqrt(jnp.maximum(p2, eps) * jnp.float32(1.0 / 6.0))
    inv_p = 1.0 / p
    b00 = dxx * inv_p
    b11 = dyy * inv_p
    b22 = dzz * inv_p
    b01 = cxy * inv_p
    b02 = cxz * inv_p
    b12 = cyz * inv_p
    detb = (b00 * (b11 * b22 - b12 * b12)
            - b01 * (b01 * b22 - b12 * b02)
            + b02 * (b01 * b12 - b11 * b02))
    r = jnp.clip(detb * 0.5, -1.0, 1.0)
    phi = _acos(r) * jnp.float32(1.0 / 3.0)
    e0 = q + 2.0 * p * jnp.cos(phi)
    e2 = q + 2.0 * p * jnp.cos(phi + jnp.float32(2.0943951023931953))
    e1 = 3.0 * q - e0 - e2
    return jnp.where(safe, e0 / e1, jnp.float32(1.0))


def _tc_body(px_ref, pxT_ref, py_ref, pyT_ref,
             er1_ref, er2_ref, idx1_ref, idx2_ref):
    iota_i = jax.lax.broadcasted_iota(jnp.int32, (_R, _N), 1)

    def er_phase(p_ref, pT_ref, er_ref):
        pT = pT_ref[0]
        pts2 = jnp.sum(pT * pT, axis=0, keepdims=True)
        pT16 = pT.astype(jnp.bfloat16)
        P = p_ref[0]
        P1 = P.astype(jnp.bfloat16)
        r1 = P - P1.astype(jnp.float32)
        P2 = r1.astype(jnp.bfloat16)
        P3 = (r1 - P2.astype(jnp.float32)).astype(jnp.bfloat16)

        def blk(i, _):
            r0 = i * _R
            a = (jnp.float32(-2.0)
                 * p_ref[0, pl.ds(r0, _R), 0:3]).astype(jnp.bfloat16)
            D = pts2 + jnp.dot(a, pT16, preferred_element_type=jnp.float32)
            Mb = _topk_mask(D).astype(jnp.bfloat16)
            S = (jnp.dot(Mb, P1, preferred_element_type=jnp.float32)
                 + jnp.dot(Mb, P2, preferred_element_type=jnp.float32)
                 + jnp.dot(Mb, P3, preferred_element_type=jnp.float32))
            er_ref[0, pl.ds(r0, _R), :] = _eigen_ratio_block(S)
            return 0

        jax.lax.fori_loop(0, _NBLK, blk, 0)

    er_phase(px_ref, pxT_ref, er1_ref)
    er_phase(py_ref, pyT_ref, er2_ref)

    def cross_phase(pa_ref, pbT_ref, idx_ref):
        pbT = pbT_ref[0]
        pts2 = jnp.sum(pbT * pbT, axis=0, keepdims=True)
        pbT16 = pbT.astype(jnp.bfloat16)

        def blk(i, _):
            r0 = i * _R
            a = (jnp.float32(-2.0)
                 * pa_ref[0, pl.ds(r0, _R), 0:3]).astype(jnp.bfloat16)
            D = pts2 + jnp.dot(a, pbT16, preferred_element_type=jnp.float32)
            m = jnp.min(D, axis=1, keepdims=True)
            key = jnp.where(D == m, iota_i, jnp.int32(2 * _N))
            amin = jnp.min(key, axis=1, keepdims=True)
            idx_ref[0, pl.ds(r0, _R), :] = amin
            return 0

        jax.lax.fori_loop(0, _NBLK, blk, 0)

    cross_phase(px_ref, pyT_ref, idx1_ref)
    cross_phase(py_ref, pxT_ref, idx2_ref)


def _sc_body(table_ref, idx_ref, era_ref, out_ref,
             idx_v, era_v, val_v, acc_v, sem):
    c = jax.lax.axis_index("c")
    s = jax.lax.axis_index("s")
    wid = s * 2 + c
    pltpu.sync_copy(idx_ref.at[wid], idx_v)
    pltpu.sync_copy(era_ref.at[wid], era_v)
    for j in range(4):
        pltpu.async_copy(table_ref.at[idx_v.at[j]],
                         val_v.at[pl.ds(j * 128, 128)], sem).wait()
    acc = jnp.zeros((16,), jnp.float32)
    for j in range(_SEG // 16):
        d = era_v[pl.ds(j * 16, 16)] - val_v[pl.ds(j * 16, 16)]
        acc = acc + d * d
    acc_v[...] = acc
    pltpu.sync_copy(acc_v, out_ref.at[wid])


def _moments(pts):
    x = pts[..., 0:1]
    y = pts[..., 1:2]
    z = pts[..., 2:3]
    zeros = jnp.zeros(pts.shape[:-1] + (7,), pts.dtype)
    return jnp.concatenate(
        [x, y, z, x * x, y * y, z * z, x * y, x * z, y * z, zeros], axis=-1)


@jax.jit
def kernel(x, y):
    B = x.shape[0]
    x3 = x[..., :3].astype(jnp.float32)
    y3 = y[..., :3].astype(jnp.float32)
    px = _moments(x3)
    py = _moments(y3)
    pxT = jnp.swapaxes(x3, 1, 2)
    pyT = jnp.swapaxes(y3, 1, 2)

    bspec_p = pl.BlockSpec((1, _N, 16), lambda b: (b, 0, 0))
    bspec_t = pl.BlockSpec((1, 3, _N), lambda b: (b, 0, 0))
    bspec_o = pl.BlockSpec((1, _N, 1), lambda b: (b, 0, 0))
    f32 = jnp.float32
    er1, er2, idx1, idx2 = pl.pallas_call(
        _tc_body,
        grid=(B,),
        in_specs=[bspec_p, bspec_t, bspec_p, bspec_t],
        out_specs=[bspec_o, bspec_o, bspec_o, bspec_o],
        out_shape=[jax.ShapeDtypeStruct((B, _N, 1), f32),
                   jax.ShapeDtypeStruct((B, _N, 1), f32),
                   jax.ShapeDtypeStruct((B, _N, 1), jnp.int32),
                   jax.ShapeDtypeStruct((B, _N, 1), jnp.int32)],
        compiler_params=pltpu.CompilerParams(
            dimension_semantics=("parallel",)),
    )(px, pxT, py, pyT)

    er1f = er1[..., 0]
    er2f = er2[..., 0]
    table = jnp.concatenate([er2f.reshape(-1), er1f.reshape(-1)])
    offs = (_N * jnp.arange(B, dtype=jnp.int32))[:, None]
    idx_all = jnp.concatenate([(idx1[..., 0] + offs).reshape(-1),
                               (idx2[..., 0] + offs + B * _N).reshape(-1)])
    era_all = jnp.concatenate([er1f.reshape(-1), er2f.reshape(-1)])

    sck = pl.kernel(
        _sc_body,
        out_type=jax.ShapeDtypeStruct((_NW, 16), f32),
        mesh=plsc.VectorSubcoreMesh(core_axis_name="c", subcore_axis_name="s"),
        scratch_types=[pltpu.VMEM((4, 128), jnp.int32),
                       pltpu.VMEM((_SEG,), f32),
                       pltpu.VMEM((_SEG,), f32),
                       pltpu.VMEM((16,), f32),
                       pltpu.SemaphoreType.DMA],
    )
    partials = sck(table,
                   idx_all.reshape(_NW, 4, 128),
                   era_all.reshape(_NW, _SEG))
    return jnp.sum(partials) * jnp.float32(0.5 / (_N * B))


# speedup vs baseline: 3.6243x; 1.0354x over previous
import functools

import jax
import jax.numpy as jnp
from jax.experimental import pallas as pl
from jax.experimental.pallas import tpu as pltpu
from jax.experimental.pallas import tpu_sc as plsc

_K = 16
_N = 4096
_R = 256
_NBLK = _N // _R
_NW = 32
_SEG = 4 * _N // _NW


def _topk_mask(D, iters):
    big = jnp.float32(1e30)
    for _ in range(iters):
        m = jnp.min(D, axis=1, keepdims=True)
        D = jnp.where(D == m, big, D)
    return (D >= jnp.float32(1e29)).astype(jnp.bfloat16)


def _acos(x):
    ax = jnp.abs(x)
    p = jnp.float32(-0.0012624911)
    p = p * ax + jnp.float32(0.0066700901)
    p = p * ax + jnp.float32(-0.0170881256)
    p = p * ax + jnp.float32(0.0308918810)
    p = p * ax + jnp.float32(-0.0501743046)
    p = p * ax + jnp.float32(0.0889789874)
    p = p * ax + jnp.float32(-0.2145988016)
    p = p * ax + jnp.float32(1.5707963050)
    a_pos = jnp.sqrt(jnp.maximum(1.0 - ax, 0.0)) * p
    return jnp.where(x >= 0, a_pos, jnp.float32(3.14159265358979) - a_pos)


def _eigen_ratio_block(S):
    k = jnp.float32(_K)
    mx = S[:, 0:1] / k
    my = S[:, 1:2] / k
    mz = S[:, 2:3] / k
    cxx = S[:, 3:4] / k - mx * mx
    cyy = S[:, 4:5] / k - my * my
    czz = S[:, 5:6] / k - mz * mz
    cxy = S[:, 6:7] / k - mx * my
    cxz = S[:, 7:8] / k - mx * mz
    cyz = S[:, 8:9] / k - my * mz

    q = (cxx + cyy + czz) * jnp.float32(1.0 / 3.0)
    p1 = cxy * cxy + cxz * cxz + cyz * cyz
    dxx = cxx - q
    dyy = cyy - q
    dzz = czz - q
    p2 = dxx * dxx + dyy * dyy + dzz * dzz + 2.0 * p1
    eps = jnp.float32(1e-30)
    safe = p2 > eps
    p = jnp.sqrt(jnp.maximum(p2, eps) * jnp.float32(1.0 / 6.0))
    inv_p = 1.0 / p
    b00 = dxx * inv_p
    b11 = dyy * inv_p
    b22 = dzz * inv_p
    b01 = cxy * inv_p
    b02 = cxz * inv_p
    b12 = cyz * inv_p
    detb = (b00 * (b11 * b22 - b12 * b12)
            - b01 * (b01 * b22 - b12 * b02)
            + b02 * (b01 * b12 - b11 * b02))
    r = jnp.clip(detb * 0.5, -1.0, 1.0)
    phi = _acos(r) * jnp.float32(1.0 / 3.0)
    e0 = q + 2.0 * p * jnp.cos(phi)
    e2 = q + 2.0 * p * jnp.cos(phi + jnp.float32(2.0943951023931953))
    e1 = 3.0 * q - e0 - e2
    return jnp.where(safe, e0 / e1, jnp.float32(1.0))


def _tc_body(px_ref, pxT_ref, py_ref, pyT_ref,
             er1_ref, er2_ref, idx1_ref, idx2_ref):
    iota_i = jax.lax.broadcasted_iota(jnp.int32, (_R, _N), 1)

    def er_phase(p_ref, pT_ref, er_ref):
        pT = pT_ref[0]
        pts2 = jnp.sum(pT * pT, axis=0, keepdims=True)
        pT16 = pT.astype(jnp.bfloat16)
        P = p_ref[0]
        P1 = P.astype(jnp.bfloat16)
        r1 = P - P1.astype(jnp.float32)
        P2 = r1.astype(jnp.bfloat16)
        P3 = (r1 - P2.astype(jnp.float32)).astype(jnp.bfloat16)

        def blk(i, _):
            r0 = i * _R
            a = (jnp.float32(-2.0)
                 * p_ref[0, pl.ds(r0, _R), 0:3]).astype(jnp.bfloat16)
            D = pts2 + jnp.dot(a, pT16, preferred_element_type=jnp.float32)
            rows = jax.lax.broadcasted_iota(jnp.int32, (_R, _N), 0) + r0
            D = jnp.where(iota_i == rows, jnp.float32(1e30), D)
            Mb = _topk_mask(D, _K - 1)
            S = (jnp.dot(Mb, P1, preferred_element_type=jnp.float32)
                 + jnp.dot(Mb, P2, preferred_element_type=jnp.float32)
                 + jnp.dot(Mb, P3, preferred_element_type=jnp.float32))
            er_ref[0, pl.ds(r0, _R), :] = _eigen_ratio_block(S)
            return 0

        jax.lax.fori_loop(0, _NBLK, blk, 0)

    er_phase(px_ref, pxT_ref, er1_ref)
    er_phase(py_ref, pyT_ref, er2_ref)

    def cross_phase(pa_ref, pbT_ref, idx_ref):
        pbT = pbT_ref[0]
        pts2 = jnp.sum(pbT * pbT, axis=0, keepdims=True)
        pbT16 = pbT.astype(jnp.bfloat16)

        def blk(i, _):
            r0 = i * _R
            a = (jnp.float32(-2.0)
                 * pa_ref[0, pl.ds(r0, _R), 0:3]).astype(jnp.bfloat16)
            D = pts2 + jnp.dot(a, pbT16, preferred_element_type=jnp.float32)
            m = jnp.min(D, axis=1, keepdims=True)
            key = jnp.where(D == m, iota_i, jnp.int32(2 * _N))
            amin = jnp.min(key, axis=1, keepdims=True)
            idx_ref[0, pl.ds(r0, _R), :] = amin
            return 0

        jax.lax.fori_loop(0, _NBLK, blk, 0)

    cross_phase(px_ref, pyT_ref, idx1_ref)
    cross_phase(py_ref, pxT_ref, idx2_ref)


def _sc_body(table_ref, idx_ref, era_ref, out_ref,
             idx_v, era_v, val_v, acc_v, sem):
    c = jax.lax.axis_index("c")
    s = jax.lax.axis_index("s")
    wid = s * 2 + c
    pltpu.sync_copy(idx_ref.at[wid], idx_v)
    pltpu.sync_copy(era_ref.at[wid], era_v)
    for j in range(4):
        pltpu.async_copy(table_ref.at[idx_v.at[j]],
                         val_v.at[pl.ds(j * 128, 128)], sem).wait()
    acc = jnp.zeros((16,), jnp.float32)
    for j in range(_SEG // 16):
        d = era_v[pl.ds(j * 16, 16)] - val_v[pl.ds(j * 16, 16)]
        acc = acc + d * d
    acc_v[...] = acc
    pltpu.sync_copy(acc_v, out_ref.at[wid])


def _moments(pts):
    x = pts[..., 0:1]
    y = pts[..., 1:2]
    z = pts[..., 2:3]
    zeros = jnp.zeros(pts.shape[:-1] + (7,), pts.dtype)
    return jnp.concatenate(
        [x, y, z, x * x, y * y, z * z, x * y, x * z, y * z, zeros], axis=-1)


@jax.jit
def kernel(x, y):
    B = x.shape[0]
    x3 = x[..., :3].astype(jnp.float32)
    y3 = y[..., :3].astype(jnp.float32)
    px = _moments(x3)
    py = _moments(y3)
    pxT = jnp.swapaxes(x3, 1, 2)
    pyT = jnp.swapaxes(y3, 1, 2)

    bspec_p = pl.BlockSpec((1, _N, 16), lambda b: (b, 0, 0))
    bspec_t = pl.BlockSpec((1, 3, _N), lambda b: (b, 0, 0))
    bspec_o = pl.BlockSpec((1, _N, 1), lambda b: (b, 0, 0))
    f32 = jnp.float32
    er1, er2, idx1, idx2 = pl.pallas_call(
        _tc_body,
        grid=(B,),
        in_specs=[bspec_p, bspec_t, bspec_p, bspec_t],
        out_specs=[bspec_o, bspec_o, bspec_o, bspec_o],
        out_shape=[jax.ShapeDtypeStruct((B, _N, 1), f32),
                   jax.ShapeDtypeStruct((B, _N, 1), f32),
                   jax.ShapeDtypeStruct((B, _N, 1), jnp.int32),
                   jax.ShapeDtypeStruct((B, _N, 1), jnp.int32)],
        compiler_params=pltpu.CompilerParams(
            dimension_semantics=("parallel",)),
    )(px, pxT, py, pyT)

    er1f = er1[..., 0]
    er2f = er2[..., 0]
    table = jnp.concatenate([er2f.reshape(-1), er1f.reshape(-1)])
    offs = (_N * jnp.arange(B, dtype=jnp.int32))[:, None]
    idx_all = jnp.concatenate([(idx1[..., 0] + offs).reshape(-1),
                               (idx2[..., 0] + offs + B * _N).reshape(-1)])
    era_all = jnp.concatenate([er1f.reshape(-1), er2f.reshape(-1)])

    sck = pl.kernel(
        _sc_body,
        out_type=jax.ShapeDtypeStruct((_NW, 16), f32),
        mesh=plsc.VectorSubcoreMesh(core_axis_name="c", subcore_axis_name="s"),
        scratch_types=[pltpu.VMEM((4, 128), jnp.int32),
                       pltpu.VMEM((_SEG,), f32),
                       pltpu.VMEM((_SEG,), f32),
                       pltpu.VMEM((16,), f32),
                       pltpu.SemaphoreType.DMA],
    )
    partials = sck(table,
                   idx_all.reshape(_NW, 4, 128),
                   era_all.reshape(_NW, _SEG))
    return jnp.sum(partials) * jnp.float32(0.5 / (_N * B))
